# baseline (speedup 1.0000x reference)
import functools

import jax
import jax.numpy as jnp
from jax import lax
from jax.experimental import pallas as pl
from jax.experimental.pallas import tpu as pltpu
from jax.experimental.pallas import tpu_sc as plsc

N_NODES = 10000
N_EDGES = 320000
D = 128

NC = 2
NS = 16
NW = NC * NS
EW = N_EDGES // NW
C = 128
K = 16
NBLK = 5
NCHUNK = NBLK * K
EWP = NCHUNK * C
PAD = EWP - EW
NDUMMY = 112
ACC_N = N_NODES + NDUMMY
ZROWS = ACC_N // NS
OUT_ROWS = 632

_MESH = plsc.VectorSubcoreMesh(core_axis_name="c", subcore_axis_name="s")


def _segsum_body(tab, srcs, dsts, zeros, out0, out1, acc,
                 srcb0, dstb0, srcb1, dstb1, buf0, buf1,
                 sem0, sem1, semi0, semi1, sems0, sems1, semz):
    cid = lax.axis_index("c")
    sid = lax.axis_index("s")
    wid = sid * NC + cid

    srcbs = (srcb0, srcb1)
    dstbs = (dstb0, dstb1)
    bufs = (buf0, buf1)
    sems = (sem0, sem1)
    semis = (semi0, semi1)
    semss = (sems0, sems1)

    def _idx_start(b, side):
        pltpu.async_copy(srcs.at[wid, pl.ds(b * K, K)], srcbs[side], semis[side])
        pltpu.async_copy(dsts.at[wid, pl.ds(b * K, K)], dstbs[side], semis[side])

    def _idx_wait(b, side):
        pltpu.make_async_copy(srcs.at[wid, pl.ds(b * K, K)], srcbs[side],
                              semis[side]).wait()
        pltpu.make_async_copy(dsts.at[wid, pl.ds(b * K, K)], dstbs[side],
                              semis[side]).wait()

    z0 = sid * ZROWS
    pltpu.async_copy(zeros.at[pl.ds(z0, ZROWS)], acc.at[pl.ds(z0, ZROWS)],
                     semz)
    _idx_start(0, 0)
    _idx_wait(0, 0)
    pltpu.async_copy(tab.at[srcb0.at[0]], buf0, sem0)
    pltpu.async_copy(tab.at[srcb0.at[1]], buf1, sem1)
    pltpu.make_async_copy(zeros.at[pl.ds(z0, ZROWS)],
                          acc.at[pl.ds(z0, ZROWS)], semz).wait()
    plsc.subcore_barrier()

    pending = []
    for b in range(NBLK):
        cs, cd = srcbs[b % 2], dstbs[b % 2]
        nside = (b + 1) % 2
        for j in range(K):
            g = b * K + j
            if j == 1 and b + 1 < NBLK:
                _idx_start(b + 1, nside)
            if g + 1 < NCHUNK and g != 0:
                if pending:
                    dref, pbuf, psem = pending.pop(0)
                    pltpu.make_async_copy(pbuf, acc.at[dref], psem).wait()
                if j + 1 < K:
                    nidx = cs.at[j + 1]
                else:
                    _idx_wait(b + 1, nside)
                    nidx = srcbs[nside].at[0]
                pltpu.async_copy(tab.at[nidx], bufs[(g + 1) % 2],
                                 sems[(g + 1) % 2])
            pltpu.make_async_copy(tab.at[cs.at[j]], bufs[g % 2],
                                  sems[g % 2]).wait()
            pltpu.async_copy(bufs[g % 2], acc.at[cd.at[j]], semss[g % 2],
                             add=True)
            pending.append((cd.at[j], bufs[g % 2], semss[g % 2]))

    for dref, pbuf, psem in pending:
        pltpu.make_async_copy(pbuf, acc.at[dref], psem).wait()

    plsc.subcore_barrier()

    o0 = sid * OUT_ROWS
    last = NS * OUT_ROWS - OUT_ROWS
    tail = N_NODES - last

    @pl.when(jnp.logical_and(cid == 0, sid < NS - 1))
    def _():
        pltpu.sync_copy(acc.at[pl.ds(o0, OUT_ROWS)], out0.at[pl.ds(o0, OUT_ROWS)])

    @pl.when(jnp.logical_and(cid == 0, sid == NS - 1))
    def _():
        pltpu.sync_copy(acc.at[pl.ds(last, tail)], out0.at[pl.ds(last, tail)])

    @pl.when(jnp.logical_and(cid == 1, sid < NS - 1))
    def _():
        pltpu.sync_copy(acc.at[pl.ds(o0, OUT_ROWS)], out1.at[pl.ds(o0, OUT_ROWS)])

    @pl.when(jnp.logical_and(cid == 1, sid == NS - 1))
    def _():
        pltpu.sync_copy(acc.at[pl.ds(last, tail)], out1.at[pl.ds(last, tail)])


_segsum_sc = pl.kernel(
    _segsum_body,
    out_type=(
        jax.ShapeDtypeStruct((N_NODES, D), jnp.float32),
        jax.ShapeDtypeStruct((N_NODES, D), jnp.float32),
    ),
    mesh=_MESH,
    scratch_types=[
        pltpu.VMEM_SHARED((ACC_N, D), jnp.float32),
        pltpu.VMEM((K, C), jnp.int32),
        pltpu.VMEM((K, C), jnp.int32),
        pltpu.VMEM((K, C), jnp.int32),
        pltpu.VMEM((K, C), jnp.int32),
        pltpu.VMEM((C, D), jnp.float32),
        pltpu.VMEM((C, D), jnp.float32),
        pltpu.SemaphoreType.DMA,
        pltpu.SemaphoreType.DMA,
        pltpu.SemaphoreType.DMA,
        pltpu.SemaphoreType.DMA,
        pltpu.SemaphoreType.DMA,
        pltpu.SemaphoreType.DMA,
        pltpu.SemaphoreType.DMA,
    ],
)


_BN = 1000
_ROW = lambda i: (i, 0)
_ZERO = lambda i: (0, 0)


def _root_body(xr, wo, br, o):
    dn = (((1,), (1,)), ((), ()))
    o[...] = lax.dot_general(xr[...], wo[...], dn,
                             preferred_element_type=jnp.float32) + br[...]


def _root_affine(x, w_root, b):
    return pl.pallas_call(
        _root_body,
        grid=(N_NODES // _BN,),
        in_specs=[
            pl.BlockSpec((_BN, D), _ROW),
            pl.BlockSpec((D, D), _ZERO),
            pl.BlockSpec((1, D), _ZERO),
        ],
        out_specs=pl.BlockSpec((_BN, D), _ROW),
        out_shape=jax.ShapeDtypeStruct((N_NODES, D), jnp.float32),
    )(x, w_root, b)


def _rel_body(p0, p1, rr, wr, o, *, relu):
    dn = (((1,), (1,)), ((), ()))
    agg = p0[...] + p1[...]
    y = lax.dot_general(agg, wr[...], dn,
                        preferred_element_type=jnp.float32) + rr[...]
    if relu:
        y = jnp.maximum(y, 0.0)
    o[...] = y


def _rel_affine(p0, p1, r, w_rel, relu):
    return pl.pallas_call(
        functools.partial(_rel_body, relu=relu),
        grid=(N_NODES // _BN,),
        in_specs=[
            pl.BlockSpec((_BN, D), _ROW),
            pl.BlockSpec((_BN, D), _ROW),
            pl.BlockSpec((_BN, D), _ROW),
            pl.BlockSpec((D, D), _ZERO),
        ],
        out_specs=pl.BlockSpec((_BN, D), _ROW),
        out_shape=jax.ShapeDtypeStruct((N_NODES, D), jnp.float32),
    )(p0, p1, r, w_rel)


def _mid_body(p0, p1, rr, wr, wo, br, ho, ro):
    dn = (((1,), (1,)), ((), ()))
    agg = p0[...] + p1[...]
    h = lax.dot_general(agg, wr[...], dn,
                        preferred_element_type=jnp.float32) + rr[...]
    h = jnp.maximum(h, 0.0)
    ho[...] = h
    ro[...] = lax.dot_general(h, wo[...], dn,
                              preferred_element_type=jnp.float32) + br[...]


def _mid_affine(p0, p1, r1, w1_rel, w2_root, b2):
    return pl.pallas_call(
        _mid_body,
        grid=(N_NODES // _BN,),
        in_specs=[
            pl.BlockSpec((_BN, D), _ROW),
            pl.BlockSpec((_BN, D), _ROW),
            pl.BlockSpec((_BN, D), _ROW),
            pl.BlockSpec((D, D), _ZERO),
            pl.BlockSpec((D, D), _ZERO),
            pl.BlockSpec((1, D), _ZERO),
        ],
        out_specs=(pl.BlockSpec((_BN, D), _ROW), pl.BlockSpec((_BN, D), _ROW)),
        out_shape=(jax.ShapeDtypeStruct((N_NODES, D), jnp.float32),
                   jax.ShapeDtypeStruct((N_NODES, D), jnp.float32)),
    )(p0, p1, r1, w1_rel, w2_root, b2)


def kernel(x, edge_index, W1_rel, b1, W1_root, W2_rel, b2, W2_root):
    src = edge_index[0].astype(jnp.int32).reshape(NW, EW)
    dst = edge_index[1].astype(jnp.int32).reshape(NW, EW)
    pad_ar = jnp.arange(PAD, dtype=jnp.int32)
    pad_src = jnp.broadcast_to((pad_ar * 89) % N_NODES, (NW, PAD))
    pad_dst = jnp.broadcast_to(N_NODES + pad_ar % NDUMMY, (NW, PAD))
    src3 = jnp.concatenate([src, pad_src], axis=1).reshape(NW, NCHUNK, C)
    dst3 = jnp.concatenate([dst, pad_dst], axis=1).reshape(NW, NCHUNK, C)

    zeros = jnp.zeros((ACC_N, D), jnp.float32)

    r1 = _root_affine(x, W1_root, b1.reshape(1, D))
    p0, p1 = _segsum_sc(x, src3, dst3, zeros)
    h, r2 = _mid_affine(p0, p1, r1, W1_rel, W2_root, b2.reshape(1, D))
    q0, q1 = _segsum_sc(h, src3, dst3, zeros)
    return _rel_affine(q0, q1, r2, W2_rel, relu=False)


# speedup vs baseline: 1.0181x; 1.0181x over previous
import functools

import jax
import jax.numpy as jnp
from jax import lax
from jax.experimental import pallas as pl
from jax.experimental.pallas import tpu as pltpu
from jax.experimental.pallas import tpu_sc as plsc

N_NODES = 10000
N_EDGES = 320000
D = 128

NC = 2
NS = 16
NW = NC * NS
EW = N_EDGES // NW
C = 128
K = 16
NBLK = 5
NCHUNK = NBLK * K
EWP = NCHUNK * C
PAD = EWP - EW
NDUMMY = 112
ACC_N = N_NODES + NDUMMY
ZROWS = ACC_N // NS
OUT_ROWS = 632

_MESH = plsc.VectorSubcoreMesh(core_axis_name="c", subcore_axis_name="s")


def _segsum_body(tab, srcs, dsts, out0, out1, acc,
                 srcb0, dstb0, srcb1, dstb1, buf0, buf1,
                 sem0, sem1, semi0, semi1, sems0, sems1):
    cid = lax.axis_index("c")
    sid = lax.axis_index("s")
    wid = sid * NC + cid

    srcbs = (srcb0, srcb1)
    dstbs = (dstb0, dstb1)
    bufs = (buf0, buf1)
    sems = (sem0, sem1)
    semis = (semi0, semi1)
    semss = (sems0, sems1)

    def _idx_start(b, side):
        pltpu.async_copy(srcs.at[wid, pl.ds(b * K, K)], srcbs[side], semis[side])
        pltpu.async_copy(dsts.at[wid, pl.ds(b * K, K)], dstbs[side], semis[side])

    def _idx_wait(b, side):
        pltpu.make_async_copy(srcs.at[wid, pl.ds(b * K, K)], srcbs[side],
                              semis[side]).wait()
        pltpu.make_async_copy(dsts.at[wid, pl.ds(b * K, K)], dstbs[side],
                              semis[side]).wait()

    _idx_start(0, 0)
    _idx_wait(0, 0)
    pltpu.async_copy(tab.at[srcb0.at[0]], buf0, sem0)

    def _zero_row(r, carry):
        for j in range(D // 16):
            buf1[r, pl.ds(j * 16, 16)] = jnp.zeros((16,), jnp.float32)
        return carry

    lax.fori_loop(0, C, _zero_row, 0)
    z0 = sid * ZROWS
    for k in range(ZROWS // C):
        pltpu.sync_copy(buf1, acc.at[pl.ds(z0 + k * C, C)])
    rem = ZROWS % C
    if rem:
        pltpu.sync_copy(buf1.at[pl.ds(0, rem)],
                        acc.at[pl.ds(z0 + (ZROWS // C) * C, rem)])
    plsc.subcore_barrier()

    pending = []
    for b in range(NBLK):
        cs, cd = srcbs[b % 2], dstbs[b % 2]
        nside = (b + 1) % 2
        for j in range(K):
            g = b * K + j
            if j == 1 and b + 1 < NBLK:
                _idx_start(b + 1, nside)
            if g + 1 < NCHUNK:
                if pending:
                    dref, pbuf, psem = pending.pop(0)
                    pltpu.make_async_copy(pbuf, acc.at[dref], psem).wait()
                if j + 1 < K:
                    nidx = cs.at[j + 1]
                else:
                    _idx_wait(b + 1, nside)
                    nidx = srcbs[nside].at[0]
                pltpu.async_copy(tab.at[nidx], bufs[(g + 1) % 2],
                                 sems[(g + 1) % 2])
            pltpu.make_async_copy(tab.at[cs.at[j]], bufs[g % 2],
                                  sems[g % 2]).wait()
            pltpu.async_copy(bufs[g % 2], acc.at[cd.at[j]], semss[g % 2],
                             add=True)
            pending.append((cd.at[j], bufs[g % 2], semss[g % 2]))

    for dref, pbuf, psem in pending:
        pltpu.make_async_copy(pbuf, acc.at[dref], psem).wait()

    plsc.subcore_barrier()

    o0 = sid * OUT_ROWS
    last = NS * OUT_ROWS - OUT_ROWS
    tail = N_NODES - last

    @pl.when(jnp.logical_and(cid == 0, sid < NS - 1))
    def _():
        pltpu.sync_copy(acc.at[pl.ds(o0, OUT_ROWS)], out0.at[pl.ds(o0, OUT_ROWS)])

    @pl.when(jnp.logical_and(cid == 0, sid == NS - 1))
    def _():
        pltpu.sync_copy(acc.at[pl.ds(last, tail)], out0.at[pl.ds(last, tail)])

    @pl.when(jnp.logical_and(cid == 1, sid < NS - 1))
    def _():
        pltpu.sync_copy(acc.at[pl.ds(o0, OUT_ROWS)], out1.at[pl.ds(o0, OUT_ROWS)])

    @pl.when(jnp.logical_and(cid == 1, sid == NS - 1))
    def _():
        pltpu.sync_copy(acc.at[pl.ds(last, tail)], out1.at[pl.ds(last, tail)])


_segsum_sc = pl.kernel(
    _segsum_body,
    out_type=(
        jax.ShapeDtypeStruct((N_NODES, D), jnp.float32),
        jax.ShapeDtypeStruct((N_NODES, D), jnp.float32),
    ),
    mesh=_MESH,
    scratch_types=[
        pltpu.VMEM_SHARED((ACC_N, D), jnp.float32),
        pltpu.VMEM((K, C), jnp.int32),
        pltpu.VMEM((K, C), jnp.int32),
        pltpu.VMEM((K, C), jnp.int32),
        pltpu.VMEM((K, C), jnp.int32),
        pltpu.VMEM((C, D), jnp.float32),
        pltpu.VMEM((C, D), jnp.float32),
        pltpu.SemaphoreType.DMA,
        pltpu.SemaphoreType.DMA,
        pltpu.SemaphoreType.DMA,
        pltpu.SemaphoreType.DMA,
        pltpu.SemaphoreType.DMA,
        pltpu.SemaphoreType.DMA,
    ],
)


_BN = 1000
_ROW = lambda i: (i, 0)
_ZERO = lambda i: (0, 0)


def _root_body(xr, wo, br, o):
    dn = (((1,), (1,)), ((), ()))
    o[...] = lax.dot_general(xr[...], wo[...], dn,
                             preferred_element_type=jnp.float32) + br[...]


def _root_affine(x, w_root, b):
    return pl.pallas_call(
        _root_body,
        grid=(N_NODES // _BN,),
        in_specs=[
            pl.BlockSpec((_BN, D), _ROW),
            pl.BlockSpec((D, D), _ZERO),
            pl.BlockSpec((1, D), _ZERO),
        ],
        out_specs=pl.BlockSpec((_BN, D), _ROW),
        out_shape=jax.ShapeDtypeStruct((N_NODES, D), jnp.float32),
    )(x, w_root, b)


def _rel_body(p0, p1, rr, wr, o, *, relu):
    dn = (((1,), (1,)), ((), ()))
    agg = p0[...] + p1[...]
    y = lax.dot_general(agg, wr[...], dn,
                        preferred_element_type=jnp.float32) + rr[...]
    if relu:
        y = jnp.maximum(y, 0.0)
    o[...] = y


def _rel_affine(p0, p1, r, w_rel, relu):
    return pl.pallas_call(
        functools.partial(_rel_body, relu=relu),
        grid=(N_NODES // _BN,),
        in_specs=[
            pl.BlockSpec((_BN, D), _ROW),
            pl.BlockSpec((_BN, D), _ROW),
            pl.BlockSpec((_BN, D), _ROW),
            pl.BlockSpec((D, D), _ZERO),
        ],
        out_specs=pl.BlockSpec((_BN, D), _ROW),
        out_shape=jax.ShapeDtypeStruct((N_NODES, D), jnp.float32),
    )(p0, p1, r, w_rel)


def _mid_body(p0, p1, rr, wr, wo, br, ho, ro):
    dn = (((1,), (1,)), ((), ()))
    agg = p0[...] + p1[...]
    h = lax.dot_general(agg, wr[...], dn,
                        preferred_element_type=jnp.float32) + rr[...]
    h = jnp.maximum(h, 0.0)
    ho[...] = h
    ro[...] = lax.dot_general(h, wo[...], dn,
                              preferred_element_type=jnp.float32) + br[...]


def _mid_affine(p0, p1, r1, w1_rel, w2_root, b2):
    return pl.pallas_call(
        _mid_body,
        grid=(N_NODES // _BN,),
        in_specs=[
            pl.BlockSpec((_BN, D), _ROW),
            pl.BlockSpec((_BN, D), _ROW),
            pl.BlockSpec((_BN, D), _ROW),
            pl.BlockSpec((D, D), _ZERO),
            pl.BlockSpec((D, D), _ZERO),
            pl.BlockSpec((1, D), _ZERO),
        ],
        out_specs=(pl.BlockSpec((_BN, D), _ROW), pl.BlockSpec((_BN, D), _ROW)),
        out_shape=(jax.ShapeDtypeStruct((N_NODES, D), jnp.float32),
                   jax.ShapeDtypeStruct((N_NODES, D), jnp.float32)),
    )(p0, p1, r1, w1_rel, w2_root, b2)


def kernel(x, edge_index, W1_rel, b1, W1_root, W2_rel, b2, W2_root):
    src = edge_index[0].astype(jnp.int32).reshape(NW, EW)
    dst = edge_index[1].astype(jnp.int32).reshape(NW, EW)
    pad_ar = jnp.arange(PAD, dtype=jnp.int32)
    pad_src = jnp.broadcast_to((pad_ar * 89) % N_NODES, (NW, PAD))
    pad_dst = jnp.broadcast_to(N_NODES + pad_ar % NDUMMY, (NW, PAD))
    src3 = jnp.concatenate([src, pad_src], axis=1).reshape(NW, NCHUNK, C)
    dst3 = jnp.concatenate([dst, pad_dst], axis=1).reshape(NW, NCHUNK, C)

    r1 = _root_affine(x, W1_root, b1.reshape(1, D))
    p0, p1 = _segsum_sc(x, src3, dst3)
    h, r2 = _mid_affine(p0, p1, r1, W1_rel, W2_root, b2.reshape(1, D))
    q0, q1 = _segsum_sc(h, src3, dst3)
    return _rel_affine(q0, q1, r2, W2_rel, relu=False)


# speedup vs baseline: 1.0332x; 1.0148x over previous
import functools

import jax
import jax.numpy as jnp
from jax import lax
from jax.experimental import pallas as pl
from jax.experimental.pallas import tpu as pltpu
from jax.experimental.pallas import tpu_sc as plsc

N_NODES = 10000
N_EDGES = 320000
D = 128

NC = 2
NS = 16
NW = NC * NS
EW = N_EDGES // NW
C = 128
K = 16
NBLK = 5
NCHUNK = NBLK * K
EWP = NCHUNK * C
PAD = EWP - EW
NDUMMY = 112
ACC_N = N_NODES + NDUMMY
ZROWS = ACC_N // NS
OUT_ROWS = 632

_MESH = plsc.VectorSubcoreMesh(core_axis_name="c", subcore_axis_name="s")


def _segsum_body(tab, srcs, dsts, out0, out1, acc,
                 srcb0, dstb0, srcb1, dstb1, buf0, buf1,
                 sem0, sem1, semi0, semi1, sems0, sems1):
    cid = lax.axis_index("c")
    sid = lax.axis_index("s")
    wid = sid * NC + cid

    srcbs = (srcb0, srcb1)
    dstbs = (dstb0, dstb1)
    bufs = (buf0, buf1)
    sems = (sem0, sem1)
    semis = (semi0, semi1)
    semss = (sems0, sems1)

    def _idx_start(b, side):
        pltpu.async_copy(srcs.at[wid, pl.ds(b * K, K)], srcbs[side], semis[side])
        pltpu.async_copy(dsts.at[wid, pl.ds(b * K, K)], dstbs[side], semis[side])

    def _idx_wait(b, side):
        pltpu.make_async_copy(srcs.at[wid, pl.ds(b * K, K)], srcbs[side],
                              semis[side]).wait()
        pltpu.make_async_copy(dsts.at[wid, pl.ds(b * K, K)], dstbs[side],
                              semis[side]).wait()

    _idx_start(0, 0)
    _idx_wait(0, 0)
    pltpu.async_copy(tab.at[srcb0.at[0]], buf0, sem0)

    def _zero_row(r, carry):
        for j in range(D // 16):
            buf1[r, pl.ds(j * 16, 16)] = jnp.zeros((16,), jnp.float32)
        return carry

    lax.fori_loop(0, C, _zero_row, 0)
    z0 = sid * ZROWS
    for k in range(ZROWS // C):
        pltpu.sync_copy(buf1, acc.at[pl.ds(z0 + k * C, C)])
    rem = ZROWS % C
    if rem:
        pltpu.sync_copy(buf1.at[pl.ds(0, rem)],
                        acc.at[pl.ds(z0 + (ZROWS // C) * C, rem)])
    plsc.subcore_barrier()

    pending = []
    for b in range(NBLK):
        cs, cd = srcbs[b % 2], dstbs[b % 2]
        nside = (b + 1) % 2
        for j in range(K):
            g = b * K + j
            if j == 1 and b + 1 < NBLK:
                _idx_start(b + 1, nside)
            if g + 1 < NCHUNK:
                if pending:
                    dref, pbuf, psem = pending.pop(0)
                    pltpu.make_async_copy(pbuf, acc.at[dref], psem).wait()
                if j + 1 < K:
                    nidx = cs.at[j + 1]
                else:
                    _idx_wait(b + 1, nside)
                    nidx = srcbs[nside].at[0]
                pltpu.async_copy(tab.at[nidx], bufs[(g + 1) % 2],
                                 sems[(g + 1) % 2])
            pltpu.make_async_copy(tab.at[cs.at[j]], bufs[g % 2],
                                  sems[g % 2]).wait()
            pltpu.async_copy(bufs[g % 2], acc.at[cd.at[j]], semss[g % 2],
                             add=True)
            pending.append((cd.at[j], bufs[g % 2], semss[g % 2]))

    for dref, pbuf, psem in pending:
        pltpu.make_async_copy(pbuf, acc.at[dref], psem).wait()

    plsc.subcore_barrier()

    o0 = sid * OUT_ROWS
    last = NS * OUT_ROWS - OUT_ROWS
    tail = N_NODES - last

    @pl.when(jnp.logical_and(cid == 0, sid < NS - 1))
    def _():
        pltpu.sync_copy(acc.at[pl.ds(o0, OUT_ROWS)], out0.at[pl.ds(o0, OUT_ROWS)])

    @pl.when(jnp.logical_and(cid == 0, sid == NS - 1))
    def _():
        pltpu.sync_copy(acc.at[pl.ds(last, tail)], out0.at[pl.ds(last, tail)])

    @pl.when(jnp.logical_and(cid == 1, sid < NS - 1))
    def _():
        pltpu.sync_copy(acc.at[pl.ds(o0, OUT_ROWS)], out1.at[pl.ds(o0, OUT_ROWS)])

    @pl.when(jnp.logical_and(cid == 1, sid == NS - 1))
    def _():
        pltpu.sync_copy(acc.at[pl.ds(last, tail)], out1.at[pl.ds(last, tail)])


_segsum_sc = pl.kernel(
    _segsum_body,
    out_type=(
        jax.ShapeDtypeStruct((N_NODES, D), jnp.float32),
        jax.ShapeDtypeStruct((N_NODES, D), jnp.float32),
    ),
    mesh=_MESH,
    scratch_types=[
        pltpu.VMEM_SHARED((ACC_N, D), jnp.float32),
        pltpu.VMEM((K, C), jnp.int32),
        pltpu.VMEM((K, C), jnp.int32),
        pltpu.VMEM((K, C), jnp.int32),
        pltpu.VMEM((K, C), jnp.int32),
        pltpu.VMEM((C, D), jnp.float32),
        pltpu.VMEM((C, D), jnp.float32),
        pltpu.SemaphoreType.DMA,
        pltpu.SemaphoreType.DMA,
        pltpu.SemaphoreType.DMA,
        pltpu.SemaphoreType.DMA,
        pltpu.SemaphoreType.DMA,
        pltpu.SemaphoreType.DMA,
    ],
)


_BN = 2000
_ROW = lambda i: (i, 0)
_ZERO = lambda i: (0, 0)


def _root_body(xr, wo, br, o):
    dn = (((1,), (1,)), ((), ()))
    o[...] = lax.dot_general(xr[...], wo[...], dn,
                             preferred_element_type=jnp.float32) + br[...]


def _root_affine(x, w_root, b):
    return pl.pallas_call(
        _root_body,
        grid=(N_NODES // _BN,),
        in_specs=[
            pl.BlockSpec((_BN, D), _ROW),
            pl.BlockSpec((D, D), _ZERO),
            pl.BlockSpec((1, D), _ZERO),
        ],
        out_specs=pl.BlockSpec((_BN, D), _ROW),
        out_shape=jax.ShapeDtypeStruct((N_NODES, D), jnp.float32),
    )(x, w_root, b)


def _rel_body(p0, p1, rr, wr, o, *, relu):
    dn = (((1,), (1,)), ((), ()))
    agg = p0[...] + p1[...]
    y = lax.dot_general(agg, wr[...], dn,
                        preferred_element_type=jnp.float32) + rr[...]
    if relu:
        y = jnp.maximum(y, 0.0)
    o[...] = y


def _rel_affine(p0, p1, r, w_rel, relu):
    return pl.pallas_call(
        functools.partial(_rel_body, relu=relu),
        grid=(N_NODES // _BN,),
        in_specs=[
            pl.BlockSpec((_BN, D), _ROW),
            pl.BlockSpec((_BN, D), _ROW),
            pl.BlockSpec((_BN, D), _ROW),
            pl.BlockSpec((D, D), _ZERO),
        ],
        out_specs=pl.BlockSpec((_BN, D), _ROW),
        out_shape=jax.ShapeDtypeStruct((N_NODES, D), jnp.float32),
    )(p0, p1, r, w_rel)


def _mid_body(p0, p1, rr, wr, wo, br, ho, ro):
    dn = (((1,), (1,)), ((), ()))
    agg = p0[...] + p1[...]
    h = lax.dot_general(agg, wr[...], dn,
                        preferred_element_type=jnp.float32) + rr[...]
    h = jnp.maximum(h, 0.0)
    ho[...] = h
    ro[...] = lax.dot_general(h, wo[...], dn,
                              preferred_element_type=jnp.float32) + br[...]


def _mid_affine(p0, p1, r1, w1_rel, w2_root, b2):
    return pl.pallas_call(
        _mid_body,
        grid=(N_NODES // _BN,),
        in_specs=[
            pl.BlockSpec((_BN, D), _ROW),
            pl.BlockSpec((_BN, D), _ROW),
            pl.BlockSpec((_BN, D), _ROW),
            pl.BlockSpec((D, D), _ZERO),
            pl.BlockSpec((D, D), _ZERO),
            pl.BlockSpec((1, D), _ZERO),
        ],
        out_specs=(pl.BlockSpec((_BN, D), _ROW), pl.BlockSpec((_BN, D), _ROW)),
        out_shape=(jax.ShapeDtypeStruct((N_NODES, D), jnp.float32),
                   jax.ShapeDtypeStruct((N_NODES, D), jnp.float32)),
    )(p0, p1, r1, w1_rel, w2_root, b2)


def kernel(x, edge_index, W1_rel, b1, W1_root, W2_rel, b2, W2_root):
    src = edge_index[0].astype(jnp.int32).reshape(NW, EW)
    dst = edge_index[1].astype(jnp.int32).reshape(NW, EW)
    pad_ar = jnp.arange(PAD, dtype=jnp.int32)
    pad_src = jnp.broadcast_to((pad_ar * 89) % N_NODES, (NW, PAD))
    pad_dst = jnp.broadcast_to(N_NODES + pad_ar % NDUMMY, (NW, PAD))
    src3 = jnp.concatenate([src, pad_src], axis=1).reshape(NW, NCHUNK, C)
    dst3 = jnp.concatenate([dst, pad_dst], axis=1).reshape(NW, NCHUNK, C)

    r1 = _root_affine(x, W1_root, b1.reshape(1, D))
    p0, p1 = _segsum_sc(x, src3, dst3)
    h, r2 = _mid_affine(p0, p1, r1, W1_rel, W2_root, b2.reshape(1, D))
    q0, q1 = _segsum_sc(h, src3, dst3)
    return _rel_affine(q0, q1, r2, W2_rel, relu=False)
